# baseline (speedup 1.0000x reference)
import functools

import jax
import jax.numpy as jnp
from jax import lax
from jax.experimental import pallas as pl
from jax.experimental.pallas import tpu as pltpu
from jax.experimental.pallas import tpu_sc as plsc

N = 10000
E = 160000
C = 128
EA = 16
SH = 25
H = 32

NC = 2
NS = 16
L = 16
NW = NC * NS
K = 128
KB = 64
ECH = 1280
EP = ECH * K
CH1 = 768
E0 = CH1 * K
E1 = EP - E0
SPARE = 128
RPT = 624

_mesh = plsc.VectorSubcoreMesh(core_axis_name="c", subcore_axis_name="s")


def _mul_inplace(acc, other, rows):
    def row(r, carry):
        for cc in range(C // L):
            sl = pl.ds(cc * L, L)
            acc[r, sl] = acc[r, sl] * other[r, sl]
        return carry
    lax.fori_loop(0, rows, row, 0)


def _make_sc_gather(ch0, trips):
    pairs = trips // 2
    nch = NW * trips

    @functools.partial(
        pl.kernel,
        out_type=(
            jax.ShapeDtypeStruct((nch * K, C), jnp.float32),
            jax.ShapeDtypeStruct((nch * K, C), jnp.float32),
        ),
        mesh=_mesh,
        scratch_types=[
            pltpu.VMEM((trips, K), jnp.int32),
            pltpu.VMEM((trips, K), jnp.int32),
            pltpu.VMEM((K, C), jnp.float32),
            pltpu.VMEM((K, C), jnp.float32),
            pltpu.VMEM((K, C), jnp.float32),
            pltpu.VMEM((K, C), jnp.float32),
            pltpu.SemaphoreType.DMA,
            pltpu.SemaphoreType.DMA,
            pltpu.SemaphoreType.DMA,
            pltpu.SemaphoreType.DMA,
            pltpu.SemaphoreType.DMA,
            pltpu.SemaphoreType.DMA,
            pltpu.SemaphoreType.DMA,
            pltpu.SemaphoreType.DMA,
        ],
    )
    def sc_gather(pre_hbm, dst_hbm, src_hbm, pred_hbm, prod_hbm,
                  dsts, srcs, bufd0, bufs0, bufd1, bufs1,
                  gd0, gs0, gd1, gs1, wd0, ws0, wd1, ws1):
        wid = lax.axis_index("s") * NC + lax.axis_index("c")
        w0 = ch0 + wid * trips
        o0 = wid * trips
        pltpu.sync_copy(dst_hbm.at[pl.ds(w0, trips)], dsts)
        pltpu.sync_copy(src_hbm.at[pl.ds(w0, trips)], srcs)

        def issue(t, bufd, bufs, gd, gs):
            pltpu.async_copy(pre_hbm.at[dsts.at[t]], bufd, gd)
            pltpu.async_copy(pre_hbm.at[srcs.at[t]], bufs, gs)

        def wait_in(t, bufd, bufs, gd, gs):
            pltpu.make_async_copy(pre_hbm.at[dsts.at[t]], bufd, gd).wait()
            pltpu.make_async_copy(pre_hbm.at[srcs.at[t]], bufs, gs).wait()

        def wait_out(buf, sem):
            pltpu.make_async_copy(pred_hbm.at[pl.ds(0, K)], buf, sem).wait()

        issue(0, bufd0, bufs0, gd0, gs0)
        issue(1, bufd1, bufs1, gd1, gs1)

        def pair(i, carry):
            t0 = 2 * i
            t1 = t0 + 1
            wait_in(t0, bufd0, bufs0, gd0, gs0)
            pltpu.async_copy(bufd0, pred_hbm.at[pl.ds((o0 + t0) * K, K)], wd0)
            _mul_inplace(bufs0, bufd0, K)
            pltpu.async_copy(bufs0, prod_hbm.at[pl.ds((o0 + t0) * K, K)], ws0)
            wait_in(t1, bufd1, bufs1, gd1, gs1)
            pltpu.async_copy(bufd1, pred_hbm.at[pl.ds((o0 + t1) * K, K)], wd1)

            @pl.when(i < pairs - 1)
            def _():
                wait_out(bufd0, wd0)
                wait_out(bufs0, ws0)
                issue(t0 + 2, bufd0, bufs0, gd0, gs0)

            _mul_inplace(bufs1, bufd1, K)
            pltpu.async_copy(bufs1, prod_hbm.at[pl.ds((o0 + t1) * K, K)], ws1)

            @pl.when(i < pairs - 1)
            def _():
                wait_out(bufd1, wd1)
                wait_out(bufs1, ws1)
                issue(t1 + 2, bufd1, bufs1, gd1, gs1)

            return carry

        lax.fori_loop(0, pairs, pair, 0)
        wait_out(bufd0, wd0)
        wait_out(bufs0, ws0)
        wait_out(bufd1, wd1)
        wait_out(bufs1, ws1)

    return sc_gather


_sca0 = _make_sc_gather(0, CH1 // NW)
_sca1 = _make_sc_gather(CH1, (ECH - CH1) // NW)


def _make_sc_scatter(dstrow0, srcrow0, trips):
    pairs = trips // 2

    @functools.partial(
        pl.kernel,
        out_type=jax.ShapeDtypeStruct((NC * N, C), jnp.float32),
        mesh=_mesh,
        scratch_types=[
            pltpu.VMEM((trips, KB), jnp.int32),
            pltpu.VMEM((trips // 2, K), jnp.int32),
            pltpu.VMEM((KB, C), jnp.float32),
            pltpu.VMEM((KB, C), jnp.float32),
            pltpu.VMEM((KB, C), jnp.float32),
            pltpu.VMEM((KB, C), jnp.float32),
            pltpu.VMEM_SHARED((N + SPARE, C), jnp.float32),
            pltpu.SemaphoreType.DMA,
            pltpu.SemaphoreType.DMA,
            pltpu.SemaphoreType.DMA,
            pltpu.SemaphoreType.DMA,
            pltpu.SemaphoreType.DMA,
            pltpu.SemaphoreType.DMA,
        ],
    )
    def sc_scatter(g_hbm, xn_hbm, dst_hbm, src_hbm, out_hbm,
                   dsts, srcs, bufx0, bufg0, bufx1, bufg1, agg,
                   gx0, gg0, gx1, gg1, ss0, ss1):
        c = lax.axis_index("c")
        s = lax.axis_index("s")
        wid = s * NC + c
        o0 = wid * trips
        pltpu.sync_copy(dst_hbm.at[pl.ds(dstrow0 + wid * trips, trips)], dsts)
        pltpu.sync_copy(src_hbm.at[pl.ds(srcrow0 + wid * (trips // 2),
                                         trips // 2)], srcs)

        def zrow(r, carry):
            for cc in range(C // L):
                bufx0[r, pl.ds(cc * L, L)] = jnp.zeros((L,), jnp.float32)
            return carry
        lax.fori_loop(0, KB, zrow, 0)
        for j in range(9):
            pltpu.sync_copy(bufx0, agg.at[pl.ds(s * RPT + j * KB, KB)])
        pltpu.sync_copy(bufx0.at[pl.ds(0, RPT - 9 * KB)],
                        agg.at[pl.ds(s * RPT + 9 * KB, RPT - 9 * KB)])

        @pl.when(s == NS - 1)
        def _():
            pltpu.sync_copy(bufx0, agg.at[pl.ds(NS * RPT, KB)])
            pltpu.sync_copy(bufx0, agg.at[pl.ds(NS * RPT + KB, KB)])
            pltpu.sync_copy(bufx0.at[pl.ds(0, 16)],
                            agg.at[pl.ds(NS * RPT + 2 * KB, 16)])

        plsc.subcore_barrier()

        def issue(t, row, col, bufx, bufg, gx, gg):
            pltpu.async_copy(xn_hbm.at[srcs.at[row, pl.ds(col, KB)]], bufx, gx)
            pltpu.async_copy(g_hbm.at[pl.ds((o0 + t) * KB, KB)], bufg, gg)

        def wait_in(t, row, col, bufx, bufg, gx, gg):
            pltpu.make_async_copy(xn_hbm.at[srcs.at[row, pl.ds(col, KB)]],
                                  bufx, gx).wait()
            pltpu.make_async_copy(g_hbm.at[pl.ds((o0 + t) * KB, KB)],
                                  bufg, gg).wait()

        def wait_sc(buf, sem):
            pltpu.make_async_copy(g_hbm.at[pl.ds(0, KB)], buf, sem).wait()

        issue(0, 0, 0, bufx0, bufg0, gx0, gg0)
        issue(1, 0, KB, bufx1, bufg1, gx1, gg1)

        def pair(i, carry):
            t0 = 2 * i
            t1 = t0 + 1
            wait_in(t0, i, 0, bufx0, bufg0, gx0, gg0)
            _mul_inplace(bufx0, bufg0, KB)
            pltpu.async_copy(bufx0, agg.at[dsts.at[t0]], ss0, add=True)
            wait_in(t1, i, KB, bufx1, bufg1, gx1, gg1)

            @pl.when(i < pairs - 1)
            def _():
                wait_sc(bufx0, ss0)
                issue(t0 + 2, i + 1, 0, bufx0, bufg0, gx0, gg0)

            _mul_inplace(bufx1, bufg1, KB)
            pltpu.async_copy(bufx1, agg.at[dsts.at[t1]], ss1, add=True)

            @pl.when(i < pairs - 1)
            def _():
                wait_sc(bufx1, ss1)
                issue(t1 + 2, i + 1, KB, bufx1, bufg1, gx1, gg1)

            return carry

        lax.fori_loop(0, pairs, pair, 0)
        wait_sc(bufx0, ss0)
        wait_sc(bufx1, ss1)

        plsc.subcore_barrier()
        pltpu.sync_copy(agg.at[pl.ds(s * RPT, RPT)],
                        out_hbm.at[pl.ds(c * N + s * RPT, RPT)])

        @pl.when(s == NS - 1)
        def _():
            pltpu.sync_copy(agg.at[pl.ds(NS * RPT, N - NS * RPT)],
                            out_hbm.at[pl.ds(c * N + NS * RPT, N - NS * RPT)])

    return sc_scatter


_scb0 = _make_sc_scatter(0, 0, E0 // (KB * NW))
_scb1 = _make_sc_scatter(E0 // KB, CH1, E1 // (KB * NW))


def _node_body(x_ref, W_pre_ref, b_pre_ref, Wg1_ref,
               bg1_ref, Wg2_ref, bg2_ref, W_node_ref, b_node_ref,
               pre_ref, xn_ref):
    x = x_ref[...]
    pre = jnp.dot(x, W_pre_ref[...], preferred_element_type=jnp.float32) + b_pre_ref[...]
    pre_ref[...] = pre
    h1 = jnp.dot(x, Wg1_ref[...], preferred_element_type=jnp.float32) + bg1_ref[...]
    h = jnp.dot(jax.nn.silu(h1), Wg2_ref[...], preferred_element_type=jnp.float32) + bg2_ref[...]
    xn_ref[...] = jnp.dot(x * h, W_node_ref[...], preferred_element_type=jnp.float32) + b_node_ref[...]


_NB = 1000


def _node_call(x, W_pre, b_pre, Wg1, bg1, Wg2, bg2, W_node, b_node):
    full = lambda r, c_: pl.BlockSpec((r, c_), lambda i: (0, 0))
    blk = lambda c_: pl.BlockSpec((_NB, c_), lambda i: (i, 0))
    return pl.pallas_call(
        _node_body,
        grid=(N // _NB,),
        in_specs=[
            blk(C),
            full(C, C), full(1, C),
            full(C, C), full(1, C), full(C, C), full(1, C),
            full(C, C), full(1, C),
        ],
        out_specs=[blk(C), blk(C)],
        out_shape=[
            jax.ShapeDtypeStruct((N, C), jnp.float32),
            jax.ShapeDtypeStruct((N, C), jnp.float32),
        ],
    )(x, W_pre, b_pre, Wg1, bg1, Wg2, bg2, W_node, b_node)


def _edge_body(pred_ref, prod_ref, ea_ref, sh_ref, WlA_ref, WlB_ref, bl1_ref,
               Wl2_ref, bl2_ref, W1_ref, b1_ref, W2_ref, b2_ref, Wsh_ref,
               g_ref):
    u = jax.nn.silu(jnp.dot(ea_ref[...], W1_ref[...], preferred_element_type=jnp.float32) + b1_ref[...])
    w_r = jnp.dot(u, W2_ref[...], preferred_element_type=jnp.float32) + b2_ref[...]
    t = (jnp.dot(pred_ref[...], WlA_ref[...], preferred_element_type=jnp.float32)
         + jnp.dot(prod_ref[...], WlB_ref[...], preferred_element_type=jnp.float32)
         + bl1_ref[...])
    w_s = jnp.dot(jax.nn.silu(t), Wl2_ref[...], preferred_element_type=jnp.float32) + bl2_ref[...]
    sh_p = jnp.dot(sh_ref[...], Wsh_ref[...], preferred_element_type=jnp.float32)
    g_ref[...] = w_r * w_s * sh_p


_EB = 2048


def _edge_call(pre_d, prod, edge_attr, edge_sh, row0, nrows, WlA, WlB, bl1,
               Wl2, bl2, W1, b1, W2, b2, W_sh):
    full = lambda r, c_: pl.BlockSpec((r, c_), lambda i: (0, 0))
    blk = lambda c_: pl.BlockSpec((_EB, c_), lambda i: (i, 0))
    off = row0 // _EB
    oblk = lambda c_: pl.BlockSpec((_EB, c_), lambda i, _o=off: (i + _o, 0))
    return pl.pallas_call(
        _edge_body,
        grid=(nrows // _EB,),
        in_specs=[
            blk(C), blk(C), oblk(EA), oblk(SH),
            full(C, H), full(C, H), full(1, H),
            full(H, C), full(1, C),
            full(EA, H), full(1, H), full(H, C), full(1, C),
            full(SH, C),
        ],
        out_specs=blk(C),
        out_shape=jax.ShapeDtypeStruct((nrows, C), jnp.float32),
    )(pre_d, prod, edge_attr, edge_sh, WlA, WlB, bl1, Wl2, bl2,
      W1, b1, W2, b2, W_sh)


def _out_body(p00_ref, p01_ref, p10_ref, p11_ref, xn_ref, W_out_ref,
              b_out_ref, o_ref):
    acc = (p00_ref[...] + p01_ref[...] + p10_ref[...] + p11_ref[...]
           + xn_ref[...])
    o_ref[...] = jnp.dot(acc, W_out_ref[...], preferred_element_type=jnp.float32) + b_out_ref[...]


def _out_call(part0, part1, xn, W_out, b_out):
    full = lambda r, c_: pl.BlockSpec((r, c_), lambda i: (0, 0))
    lo = pl.BlockSpec((_NB, C), lambda i: (i, 0))
    hi = pl.BlockSpec((_NB, C), lambda i: (i + N // _NB, 0))
    return pl.pallas_call(
        _out_body,
        grid=(N // _NB,),
        in_specs=[lo, hi, lo, hi, lo, full(C, C), full(1, C)],
        out_specs=lo,
        out_shape=jax.ShapeDtypeStruct((N, C), jnp.float32),
    )(part0, part0, part1, part1, xn, W_out, b_out)


def kernel(x, edge_index, edge_attr, edge_sh, W_pre, b_pre, Wg1, bg1, Wg2,
           bg2, W_node, b_node, W1, b1, W2, b2, Wl1, bl1, Wl2, bl2, W_sh,
           W_out, b_out):
    dst = edge_index[0]
    src = edge_index[1]
    WlA = Wl1[:C]
    WlB = Wl1[C:]

    pad = EP - E
    spread = jnp.arange(pad, dtype=dst.dtype)
    srcp = jnp.concatenate([src, spread % N]).reshape(ECH, K)
    dstA = jnp.concatenate([dst, spread % N]).reshape(ECH, K)
    dstB = jnp.concatenate([dst, N + (spread % SPARE)]).reshape(EP // KB, KB)
    eap = jnp.concatenate([edge_attr, jnp.zeros((pad, EA), edge_attr.dtype)])
    shp = jnp.concatenate([edge_sh, jnp.zeros((pad, SH), edge_sh.dtype)])

    pre_x, xn = _node_call(
        x, W_pre, b_pre.reshape(1, C),
        Wg1, bg1.reshape(1, C), Wg2, bg2.reshape(1, C),
        W_node, b_node.reshape(1, C))

    pre_d0, prod0 = _sca0(pre_x, dstA, srcp)
    pre_d1, prod1 = _sca1(pre_x, dstA, srcp)

    mlp_w = (WlA, WlB, bl1.reshape(1, H), Wl2, bl2.reshape(1, C),
             W1, b1.reshape(1, H), W2, b2.reshape(1, C), W_sh)
    g0 = _edge_call(pre_d0, prod0, eap, shp, 0, E0, *mlp_w)
    g1 = _edge_call(pre_d1, prod1, eap, shp, E0, E1, *mlp_w)

    part0 = _scb0(g0, xn, dstB, srcp)
    part1 = _scb1(g1, xn, dstB, srcp)

    return _out_call(part0, part1, xn, W_out, b_out.reshape(1, C))


# speedup vs baseline: 1.1176x; 1.1176x over previous
import functools

import jax
import jax.numpy as jnp
from jax import lax
from jax.experimental import pallas as pl
from jax.experimental.pallas import tpu as pltpu
from jax.experimental.pallas import tpu_sc as plsc

N = 10000
E = 160000
C = 128
EA = 16
SH = 25
H = 32

NC = 2
NS = 16
L = 16
NW = NC * NS
K = 128
TRIPS = 40
PAIRS = TRIPS // 2
ECH = NW * TRIPS
EP = ECH * K
KB = 64
TRIPS_B = EP // (KB * NW)
PAIRS_B = TRIPS_B // 2
ECH_B = NW * TRIPS_B
RPT = 624

_mesh = plsc.VectorSubcoreMesh(core_axis_name="c", subcore_axis_name="s")


def _mul_inplace(acc, other, rows):
    def row(r, carry):
        for cc in range(C // L):
            sl = pl.ds(cc * L, L)
            acc[r, sl] = acc[r, sl] * other[r, sl]
        return carry
    lax.fori_loop(0, rows, row, 0)


@functools.partial(
    pl.kernel,
    out_type=(
        jax.ShapeDtypeStruct((EP, C), jnp.float32),
        jax.ShapeDtypeStruct((EP, C), jnp.float32),
    ),
    mesh=_mesh,
    scratch_types=[
        pltpu.VMEM((TRIPS, K), jnp.int32),
        pltpu.VMEM((TRIPS, K), jnp.int32),
        pltpu.VMEM((K, C), jnp.float32),
        pltpu.VMEM((K, C), jnp.float32),
        pltpu.VMEM((K, C), jnp.float32),
        pltpu.VMEM((K, C), jnp.float32),
        pltpu.SemaphoreType.DMA,
        pltpu.SemaphoreType.DMA,
        pltpu.SemaphoreType.DMA,
        pltpu.SemaphoreType.DMA,
        pltpu.SemaphoreType.DMA,
        pltpu.SemaphoreType.DMA,
        pltpu.SemaphoreType.DMA,
        pltpu.SemaphoreType.DMA,
    ],
)
def _sc_gather(pre_hbm, dst_hbm, src_hbm, pred_hbm, prod_hbm,
               dsts, srcs, bufd0, bufs0, bufd1, bufs1,
               gd0, gs0, gd1, gs1, wd0, ws0, wd1, ws1):
    wid = lax.axis_index("s") * NC + lax.axis_index("c")
    w0 = wid * TRIPS
    pltpu.sync_copy(dst_hbm.at[pl.ds(w0, TRIPS)], dsts)
    pltpu.sync_copy(src_hbm.at[pl.ds(w0, TRIPS)], srcs)

    def issue(t, bufd, bufs, gd, gs):
        pltpu.async_copy(pre_hbm.at[dsts.at[t]], bufd, gd)
        pltpu.async_copy(pre_hbm.at[srcs.at[t]], bufs, gs)

    def wait_in(t, bufd, bufs, gd, gs):
        pltpu.make_async_copy(pre_hbm.at[dsts.at[t]], bufd, gd).wait()
        pltpu.make_async_copy(pre_hbm.at[srcs.at[t]], bufs, gs).wait()

    def wait_out(buf, sem):
        pltpu.make_async_copy(pred_hbm.at[pl.ds(0, K)], buf, sem).wait()

    issue(0, bufd0, bufs0, gd0, gs0)
    issue(1, bufd1, bufs1, gd1, gs1)

    def pair(i, carry):
        t0 = 2 * i
        t1 = t0 + 1
        wait_in(t0, bufd0, bufs0, gd0, gs0)
        pltpu.async_copy(bufd0, pred_hbm.at[pl.ds((w0 + t0) * K, K)], wd0)
        _mul_inplace(bufs0, bufd0, K)
        pltpu.async_copy(bufs0, prod_hbm.at[pl.ds((w0 + t0) * K, K)], ws0)
        wait_in(t1, bufd1, bufs1, gd1, gs1)
        pltpu.async_copy(bufd1, pred_hbm.at[pl.ds((w0 + t1) * K, K)], wd1)

        @pl.when(i < PAIRS - 1)
        def _():
            wait_out(bufd0, wd0)
            wait_out(bufs0, ws0)
            issue(t0 + 2, bufd0, bufs0, gd0, gs0)

        _mul_inplace(bufs1, bufd1, K)
        pltpu.async_copy(bufs1, prod_hbm.at[pl.ds((w0 + t1) * K, K)], ws1)

        @pl.when(i < PAIRS - 1)
        def _():
            wait_out(bufd1, wd1)
            wait_out(bufs1, ws1)
            issue(t1 + 2, bufd1, bufs1, gd1, gs1)

        return carry

    lax.fori_loop(0, PAIRS, pair, 0)
    wait_out(bufd0, wd0)
    wait_out(bufs0, ws0)
    wait_out(bufd1, wd1)
    wait_out(bufs1, ws1)


@functools.partial(
    pl.kernel,
    out_type=jax.ShapeDtypeStruct((NC * N, C), jnp.float32),
    mesh=_mesh,
    scratch_types=[
        pltpu.VMEM((TRIPS_B, KB), jnp.int32),
        pltpu.VMEM((TRIPS_B // 2, K), jnp.int32),
        pltpu.VMEM((KB, C), jnp.float32),
        pltpu.VMEM((KB, C), jnp.float32),
        pltpu.VMEM((KB, C), jnp.float32),
        pltpu.VMEM((KB, C), jnp.float32),
        pltpu.VMEM_SHARED((N + 128, C), jnp.float32),
        pltpu.SemaphoreType.DMA,
        pltpu.SemaphoreType.DMA,
        pltpu.SemaphoreType.DMA,
        pltpu.SemaphoreType.DMA,
        pltpu.SemaphoreType.DMA,
        pltpu.SemaphoreType.DMA,
    ],
)
def _sc_scatter(g_hbm, xn_hbm, dst_hbm, src_hbm, out_hbm,
                dsts, srcs, bufx0, bufg0, bufx1, bufg1, agg,
                gx0, gg0, gx1, gg1, ss0, ss1):
    c = lax.axis_index("c")
    s = lax.axis_index("s")
    wid = s * NC + c
    w0 = wid * TRIPS_B
    pltpu.sync_copy(dst_hbm.at[pl.ds(w0, TRIPS_B)], dsts)
    pltpu.sync_copy(src_hbm.at[pl.ds(wid * (TRIPS_B // 2), TRIPS_B // 2)], srcs)

    def zrow(r, carry):
        for cc in range(C // L):
            bufx0[r, pl.ds(cc * L, L)] = jnp.zeros((L,), jnp.float32)
        return carry
    lax.fori_loop(0, KB, zrow, 0)
    for j in range(9):
        pltpu.sync_copy(bufx0, agg.at[pl.ds(s * RPT + j * KB, KB)])
    pltpu.sync_copy(bufx0.at[pl.ds(0, RPT - 9 * KB)],
                    agg.at[pl.ds(s * RPT + 9 * KB, RPT - 9 * KB)])

    @pl.when(s == NS - 1)
    def _():
        pltpu.sync_copy(bufx0, agg.at[pl.ds(NS * RPT, KB)])
        pltpu.sync_copy(bufx0, agg.at[pl.ds(NS * RPT + KB, KB)])
        pltpu.sync_copy(bufx0.at[pl.ds(0, 16)],
                        agg.at[pl.ds(NS * RPT + 2 * KB, 16)])

    plsc.subcore_barrier()

    def issue(t, row, col, bufx, bufg, gx, gg):
        pltpu.async_copy(xn_hbm.at[srcs.at[row, pl.ds(col, KB)]], bufx, gx)
        pltpu.async_copy(g_hbm.at[pl.ds((w0 + t) * KB, KB)], bufg, gg)

    def wait_in(t, row, col, bufx, bufg, gx, gg):
        pltpu.make_async_copy(xn_hbm.at[srcs.at[row, pl.ds(col, KB)]], bufx, gx).wait()
        pltpu.make_async_copy(g_hbm.at[pl.ds((w0 + t) * KB, KB)], bufg, gg).wait()

    def wait_sc(buf, sem):
        pltpu.make_async_copy(g_hbm.at[pl.ds(0, KB)], buf, sem).wait()

    issue(0, 0, 0, bufx0, bufg0, gx0, gg0)
    issue(1, 0, KB, bufx1, bufg1, gx1, gg1)

    def pair(i, carry):
        t0 = 2 * i
        t1 = t0 + 1
        wait_in(t0, i, 0, bufx0, bufg0, gx0, gg0)
        _mul_inplace(bufx0, bufg0, KB)
        pltpu.async_copy(bufx0, agg.at[dsts.at[t0]], ss0, add=True)
        wait_in(t1, i, KB, bufx1, bufg1, gx1, gg1)

        @pl.when(i < PAIRS_B - 1)
        def _():
            wait_sc(bufx0, ss0)
            issue(t0 + 2, i + 1, 0, bufx0, bufg0, gx0, gg0)

        _mul_inplace(bufx1, bufg1, KB)
        pltpu.async_copy(bufx1, agg.at[dsts.at[t1]], ss1, add=True)

        @pl.when(i < PAIRS_B - 1)
        def _():
            wait_sc(bufx1, ss1)
            issue(t1 + 2, i + 1, KB, bufx1, bufg1, gx1, gg1)

        return carry

    lax.fori_loop(0, PAIRS_B, pair, 0)
    wait_sc(bufx0, ss0)
    wait_sc(bufx1, ss1)

    plsc.subcore_barrier()
    pltpu.sync_copy(agg.at[pl.ds(s * RPT, RPT)],
                    out_hbm.at[pl.ds(c * N + s * RPT, RPT)])

    @pl.when(s == NS - 1)
    def _():
        pltpu.sync_copy(agg.at[pl.ds(NS * RPT, N - NS * RPT)],
                        out_hbm.at[pl.ds(c * N + NS * RPT, N - NS * RPT)])


def _node_body(x_ref, W_pre_ref, b_pre_ref, Wg1_ref,
               bg1_ref, Wg2_ref, bg2_ref, W_node_ref, b_node_ref,
               pre_ref, xn_ref):
    x = x_ref[...]
    pre = jnp.dot(x, W_pre_ref[...], preferred_element_type=jnp.float32) + b_pre_ref[...]
    pre_ref[...] = pre
    h1 = jnp.dot(x, Wg1_ref[...], preferred_element_type=jnp.float32) + bg1_ref[...]
    h = jnp.dot(jax.nn.silu(h1), Wg2_ref[...], preferred_element_type=jnp.float32) + bg2_ref[...]
    xn_ref[...] = jnp.dot(x * h, W_node_ref[...], preferred_element_type=jnp.float32) + b_node_ref[...]


_NB = 1000


def _node_call(x, W_pre, b_pre, Wg1, bg1, Wg2, bg2, W_node, b_node):
    full = lambda r, c_: pl.BlockSpec((r, c_), lambda i: (0, 0))
    blk = lambda c_: pl.BlockSpec((_NB, c_), lambda i: (i, 0))
    return pl.pallas_call(
        _node_body,
        grid=(N // _NB,),
        in_specs=[
            blk(C),
            full(C, C), full(1, C),
            full(C, C), full(1, C), full(C, C), full(1, C),
            full(C, C), full(1, C),
        ],
        out_specs=[blk(C), blk(C)],
        out_shape=[
            jax.ShapeDtypeStruct((N, C), jnp.float32),
            jax.ShapeDtypeStruct((N, C), jnp.float32),
        ],
    )(x, W_pre, b_pre, Wg1, bg1, Wg2, bg2, W_node, b_node)


def _edge_body(pred_ref, prod_ref, ea_ref, sh_ref, WlA_ref, WlB_ref, bl1_ref,
               Wl2_ref, bl2_ref, W1_ref, b1_ref, W2_ref, b2_ref, Wsh_ref,
               g_ref):
    u = jax.nn.silu(jnp.dot(ea_ref[...], W1_ref[...], preferred_element_type=jnp.float32) + b1_ref[...])
    w_r = jnp.dot(u, W2_ref[...], preferred_element_type=jnp.float32) + b2_ref[...]
    t = (jnp.dot(pred_ref[...], WlA_ref[...], preferred_element_type=jnp.float32)
         + jnp.dot(prod_ref[...], WlB_ref[...], preferred_element_type=jnp.float32)
         + bl1_ref[...])
    w_s = jnp.dot(jax.nn.silu(t), Wl2_ref[...], preferred_element_type=jnp.float32) + bl2_ref[...]
    sh_p = jnp.dot(sh_ref[...], Wsh_ref[...], preferred_element_type=jnp.float32)
    g_ref[...] = w_r * w_s * sh_p


_EB = 4000


def _edge_call(pre_d, prod, edge_attr, edge_sh, WlA, WlB, bl1, Wl2, bl2,
               W1, b1, W2, b2, W_sh):
    full = lambda r, c_: pl.BlockSpec((r, c_), lambda i: (0, 0))
    blk = lambda c_: pl.BlockSpec((_EB, c_), lambda i: (i, 0))
    return pl.pallas_call(
        _edge_body,
        grid=(E // _EB,),
        in_specs=[
            blk(C), blk(C), blk(EA), blk(SH),
            full(C, H), full(C, H), full(1, H),
            full(H, C), full(1, C),
            full(EA, H), full(1, H), full(H, C), full(1, C),
            full(SH, C),
        ],
        out_specs=blk(C),
        out_shape=jax.ShapeDtypeStruct((EP, C), jnp.float32),
    )(pre_d, prod, edge_attr, edge_sh, WlA, WlB, bl1, Wl2, bl2,
      W1, b1, W2, b2, W_sh)


def _out_body(p0_ref, p1_ref, xn_ref, W_out_ref, b_out_ref, o_ref):
    acc = p0_ref[...] + p1_ref[...] + xn_ref[...]
    o_ref[...] = jnp.dot(acc, W_out_ref[...], preferred_element_type=jnp.float32) + b_out_ref[...]


def _out_call(part, xn, W_out, b_out):
    full = lambda r, c_: pl.BlockSpec((r, c_), lambda i: (0, 0))
    return pl.pallas_call(
        _out_body,
        grid=(N // _NB,),
        in_specs=[
            pl.BlockSpec((_NB, C), lambda i: (i, 0)),
            pl.BlockSpec((_NB, C), lambda i: (i + N // _NB, 0)),
            pl.BlockSpec((_NB, C), lambda i: (i, 0)),
            full(C, C), full(1, C),
        ],
        out_specs=pl.BlockSpec((_NB, C), lambda i: (i, 0)),
        out_shape=jax.ShapeDtypeStruct((N, C), jnp.float32),
    )(part, part, xn, W_out, b_out)


def kernel(x, edge_index, edge_attr, edge_sh, W_pre, b_pre, Wg1, bg1, Wg2,
           bg2, W_node, b_node, W1, b1, W2, b2, Wl1, bl1, Wl2, bl2, W_sh,
           W_out, b_out):
    dst = edge_index[0]
    src = edge_index[1]
    WlA = Wl1[:C]
    WlB = Wl1[C:]

    pad = EP - E
    spread = jnp.arange(pad, dtype=dst.dtype)
    srcp = jnp.concatenate([src, spread % N]).reshape(ECH, K)
    dstA = jnp.concatenate([dst, spread % N]).reshape(ECH, K)
    dstB = jnp.concatenate([dst, N + (spread % 128)]).reshape(ECH_B, KB)

    pre_x, xn = _node_call(
        x, W_pre, b_pre.reshape(1, C),
        Wg1, bg1.reshape(1, C), Wg2, bg2.reshape(1, C),
        W_node, b_node.reshape(1, C))

    pre_d, prod = _sc_gather(pre_x, dstA, srcp)

    g = _edge_call(pre_d, prod, edge_attr, edge_sh, WlA, WlB,
                   bl1.reshape(1, H), Wl2, bl2.reshape(1, C),
                   W1, b1.reshape(1, H), W2, b2.reshape(1, C), W_sh)

    part = _sc_scatter(g, xn, dstB, srcp)

    return _out_call(part, xn, W_out, b_out.reshape(1, C))


# speedup vs baseline: 1.1290x; 1.0102x over previous
import functools

import jax
import jax.numpy as jnp
from jax import lax
from jax.experimental import pallas as pl
from jax.experimental.pallas import tpu as pltpu
from jax.experimental.pallas import tpu_sc as plsc

N = 10000
E = 160000
C = 128
EA = 16
SH = 25
H = 32

NC = 2
NS = 16
L = 16
NW = NC * NS
K = 128
TRIPS = 40
PAIRS = TRIPS // 2
ECH = NW * TRIPS
EP = ECH * K
KB = 64
TRIPS_B = EP // (KB * NW)
PAIRS_B = TRIPS_B // 2
ECH_B = NW * TRIPS_B
RPT = 624

_mesh = plsc.VectorSubcoreMesh(core_axis_name="c", subcore_axis_name="s")


def _mul_inplace(acc, other, rows):
    def row(r, carry):
        for cc in range(C // L):
            sl = pl.ds(cc * L, L)
            acc[r, sl] = acc[r, sl] * other[r, sl]
        return carry
    lax.fori_loop(0, rows, row, 0)


@functools.partial(
    pl.kernel,
    out_type=(
        jax.ShapeDtypeStruct((EP, C), jnp.float32),
        jax.ShapeDtypeStruct((EP, C), jnp.float32),
    ),
    mesh=_mesh,
    scratch_types=[
        pltpu.VMEM((TRIPS, K), jnp.int32),
        pltpu.VMEM((TRIPS, K), jnp.int32),
        pltpu.VMEM((K, C), jnp.float32),
        pltpu.VMEM((K, C), jnp.float32),
        pltpu.VMEM((K, C), jnp.float32),
        pltpu.VMEM((K, C), jnp.float32),
        pltpu.SemaphoreType.DMA,
        pltpu.SemaphoreType.DMA,
        pltpu.SemaphoreType.DMA,
        pltpu.SemaphoreType.DMA,
        pltpu.SemaphoreType.DMA,
        pltpu.SemaphoreType.DMA,
        pltpu.SemaphoreType.DMA,
        pltpu.SemaphoreType.DMA,
    ],
)
def _sc_gather(pre_hbm, dst_hbm, src_hbm, pred_hbm, prod_hbm,
               dsts, srcs, bufd0, bufs0, bufd1, bufs1,
               gd0, gs0, gd1, gs1, wd0, ws0, wd1, ws1):
    wid = lax.axis_index("s") * NC + lax.axis_index("c")
    w0 = wid * TRIPS
    pltpu.sync_copy(dst_hbm.at[pl.ds(w0, TRIPS)], dsts)
    pltpu.sync_copy(src_hbm.at[pl.ds(w0, TRIPS)], srcs)

    def issue(t, bufd, bufs, gd, gs):
        pltpu.async_copy(pre_hbm.at[dsts.at[t]], bufd, gd)
        pltpu.async_copy(pre_hbm.at[srcs.at[t]], bufs, gs)

    def wait_in(t, bufd, bufs, gd, gs):
        pltpu.make_async_copy(pre_hbm.at[dsts.at[t]], bufd, gd).wait()
        pltpu.make_async_copy(pre_hbm.at[srcs.at[t]], bufs, gs).wait()

    def wait_out(buf, sem):
        pltpu.make_async_copy(pred_hbm.at[pl.ds(0, K)], buf, sem).wait()

    issue(0, bufd0, bufs0, gd0, gs0)
    issue(1, bufd1, bufs1, gd1, gs1)

    def pair(i, carry):
        t0 = 2 * i
        t1 = t0 + 1
        wait_in(t0, bufd0, bufs0, gd0, gs0)
        pltpu.async_copy(bufd0, pred_hbm.at[pl.ds((w0 + t0) * K, K)], wd0)
        _mul_inplace(bufs0, bufd0, K)
        pltpu.async_copy(bufs0, prod_hbm.at[pl.ds((w0 + t0) * K, K)], ws0)
        wait_in(t1, bufd1, bufs1, gd1, gs1)
        pltpu.async_copy(bufd1, pred_hbm.at[pl.ds((w0 + t1) * K, K)], wd1)

        @pl.when(i < PAIRS - 1)
        def _():
            wait_out(bufd0, wd0)
            wait_out(bufs0, ws0)
            issue(t0 + 2, bufd0, bufs0, gd0, gs0)

        _mul_inplace(bufs1, bufd1, K)
        pltpu.async_copy(bufs1, prod_hbm.at[pl.ds((w0 + t1) * K, K)], ws1)

        @pl.when(i < PAIRS - 1)
        def _():
            wait_out(bufd1, wd1)
            wait_out(bufs1, ws1)
            issue(t1 + 2, bufd1, bufs1, gd1, gs1)

        return carry

    lax.fori_loop(0, PAIRS, pair, 0)
    wait_out(bufd0, wd0)
    wait_out(bufs0, ws0)
    wait_out(bufd1, wd1)
    wait_out(bufs1, ws1)


@functools.partial(
    pl.kernel,
    out_type=jax.ShapeDtypeStruct((NC * N, C), jnp.float32),
    mesh=_mesh,
    scratch_types=[
        pltpu.VMEM((TRIPS_B, KB), jnp.int32),
        pltpu.VMEM((TRIPS_B // 2, K), jnp.int32),
        pltpu.VMEM((KB, C), jnp.float32),
        pltpu.VMEM((KB, C), jnp.float32),
        pltpu.VMEM((KB, C), jnp.float32),
        pltpu.VMEM((KB, C), jnp.float32),
        pltpu.VMEM_SHARED((N + 128, C), jnp.float32),
        pltpu.SemaphoreType.DMA,
        pltpu.SemaphoreType.DMA,
        pltpu.SemaphoreType.DMA,
        pltpu.SemaphoreType.DMA,
        pltpu.SemaphoreType.DMA,
        pltpu.SemaphoreType.DMA,
    ],
)
def _sc_scatter(g_hbm, xn_hbm, dst_hbm, src_hbm, out_hbm,
                dsts, srcs, bufx0, bufg0, bufx1, bufg1, agg,
                gx0, gg0, gx1, gg1, ss0, ss1):
    c = lax.axis_index("c")
    s = lax.axis_index("s")
    wid = s * NC + c
    w0 = wid * TRIPS_B
    pltpu.sync_copy(dst_hbm.at[pl.ds(w0, TRIPS_B)], dsts)
    pltpu.sync_copy(src_hbm.at[pl.ds(wid * (TRIPS_B // 2), TRIPS_B // 2)], srcs)

    def zrow(r, carry):
        for cc in range(C // L):
            bufx0[r, pl.ds(cc * L, L)] = jnp.zeros((L,), jnp.float32)
        return carry
    lax.fori_loop(0, KB, zrow, 0)
    for j in range(9):
        pltpu.sync_copy(bufx0, agg.at[pl.ds(s * RPT + j * KB, KB)])
    pltpu.sync_copy(bufx0.at[pl.ds(0, RPT - 9 * KB)],
                    agg.at[pl.ds(s * RPT + 9 * KB, RPT - 9 * KB)])

    @pl.when(s == NS - 1)
    def _():
        pltpu.sync_copy(bufx0, agg.at[pl.ds(NS * RPT, KB)])
        pltpu.sync_copy(bufx0, agg.at[pl.ds(NS * RPT + KB, KB)])
        pltpu.sync_copy(bufx0.at[pl.ds(0, 16)],
                        agg.at[pl.ds(NS * RPT + 2 * KB, 16)])

    plsc.subcore_barrier()

    def issue(t, row, col, bufx, bufg, gx, gg):
        pltpu.async_copy(xn_hbm.at[srcs.at[row, pl.ds(col, KB)]], bufx, gx)
        pltpu.async_copy(g_hbm.at[pl.ds((w0 + t) * KB, KB)], bufg, gg)

    def wait_in(t, row, col, bufx, bufg, gx, gg):
        pltpu.make_async_copy(xn_hbm.at[srcs.at[row, pl.ds(col, KB)]], bufx, gx).wait()
        pltpu.make_async_copy(g_hbm.at[pl.ds((w0 + t) * KB, KB)], bufg, gg).wait()

    def wait_sc(buf, sem):
        pltpu.make_async_copy(g_hbm.at[pl.ds(0, KB)], buf, sem).wait()

    issue(0, 0, 0, bufx0, bufg0, gx0, gg0)
    issue(1, 0, KB, bufx1, bufg1, gx1, gg1)

    def pair(i, carry):
        t0 = 2 * i
        t1 = t0 + 1
        wait_in(t0, i, 0, bufx0, bufg0, gx0, gg0)
        _mul_inplace(bufx0, bufg0, KB)
        pltpu.async_copy(bufx0, agg.at[dsts.at[t0]], ss0, add=True)
        wait_in(t1, i, KB, bufx1, bufg1, gx1, gg1)

        @pl.when(i < PAIRS_B - 1)
        def _():
            wait_sc(bufx0, ss0)
            issue(t0 + 2, i + 1, 0, bufx0, bufg0, gx0, gg0)

        _mul_inplace(bufx1, bufg1, KB)
        pltpu.async_copy(bufx1, agg.at[dsts.at[t1]], ss1, add=True)

        @pl.when(i < PAIRS_B - 1)
        def _():
            wait_sc(bufx1, ss1)
            issue(t1 + 2, i + 1, KB, bufx1, bufg1, gx1, gg1)

        return carry

    lax.fori_loop(0, PAIRS_B, pair, 0)
    wait_sc(bufx0, ss0)
    wait_sc(bufx1, ss1)

    plsc.subcore_barrier()
    pltpu.sync_copy(agg.at[pl.ds(s * RPT, RPT)],
                    out_hbm.at[pl.ds(c * N + s * RPT, RPT)])

    @pl.when(s == NS - 1)
    def _():
        pltpu.sync_copy(agg.at[pl.ds(NS * RPT, N - NS * RPT)],
                        out_hbm.at[pl.ds(c * N + NS * RPT, N - NS * RPT)])


def _node_body(x_ref, W_pre_ref, b_pre_ref, Wg1_ref,
               bg1_ref, Wg2_ref, bg2_ref, W_node_ref, b_node_ref,
               pre_ref, xn_ref):
    x = x_ref[...]
    pre = jnp.dot(x, W_pre_ref[...], preferred_element_type=jnp.float32) + b_pre_ref[...]
    pre_ref[...] = pre
    h1 = jnp.dot(x, Wg1_ref[...], preferred_element_type=jnp.float32) + bg1_ref[...]
    h = jnp.dot(jax.nn.silu(h1), Wg2_ref[...], preferred_element_type=jnp.float32) + bg2_ref[...]
    xn_ref[...] = jnp.dot(x * h, W_node_ref[...], preferred_element_type=jnp.float32) + b_node_ref[...]


_NB = 1000


def _node_call(x, W_pre, b_pre, Wg1, bg1, Wg2, bg2, W_node, b_node):
    full = lambda r, c_: pl.BlockSpec((r, c_), lambda i: (0, 0))
    blk = lambda c_: pl.BlockSpec((_NB, c_), lambda i: (i, 0))
    return pl.pallas_call(
        _node_body,
        grid=(N // _NB,),
        in_specs=[
            blk(C),
            full(C, C), full(1, C),
            full(C, C), full(1, C), full(C, C), full(1, C),
            full(C, C), full(1, C),
        ],
        out_specs=[blk(C), blk(C)],
        out_shape=[
            jax.ShapeDtypeStruct((N, C), jnp.float32),
            jax.ShapeDtypeStruct((N, C), jnp.float32),
        ],
    )(x, W_pre, b_pre, Wg1, bg1, Wg2, bg2, W_node, b_node)


def _edge_body(pred_ref, prod_ref, ea_ref, sh_ref, WlA_ref, WlB_ref, bl1_ref,
               Wl2_ref, bl2_ref, W1_ref, b1_ref, W2_ref, b2_ref, Wsh_ref,
               g_ref):
    u = jax.nn.silu(jnp.dot(ea_ref[...], W1_ref[...], preferred_element_type=jnp.float32) + b1_ref[...])
    w_r = jnp.dot(u, W2_ref[...], preferred_element_type=jnp.float32) + b2_ref[...]
    t = (jnp.dot(pred_ref[...], WlA_ref[...], preferred_element_type=jnp.float32)
         + jnp.dot(prod_ref[...], WlB_ref[...], preferred_element_type=jnp.float32)
         + bl1_ref[...])
    w_s = jnp.dot(jax.nn.silu(t), Wl2_ref[...], preferred_element_type=jnp.float32) + bl2_ref[...]
    sh_p = jnp.dot(sh_ref[...], Wsh_ref[...], preferred_element_type=jnp.float32)
    g_ref[...] = w_r * w_s * sh_p


_EB = 8000


def _edge_call(pre_d, prod, edge_attr, edge_sh, WlA, WlB, bl1, Wl2, bl2,
               W1, b1, W2, b2, W_sh):
    full = lambda r, c_: pl.BlockSpec((r, c_), lambda i: (0, 0))
    blk = lambda c_: pl.BlockSpec((_EB, c_), lambda i: (i, 0))
    return pl.pallas_call(
        _edge_body,
        grid=(E // _EB,),
        in_specs=[
            blk(C), blk(C), blk(EA), blk(SH),
            full(C, H), full(C, H), full(1, H),
            full(H, C), full(1, C),
            full(EA, H), full(1, H), full(H, C), full(1, C),
            full(SH, C),
        ],
        out_specs=blk(C),
        out_shape=jax.ShapeDtypeStruct((EP, C), jnp.float32),
    )(pre_d, prod, edge_attr, edge_sh, WlA, WlB, bl1, Wl2, bl2,
      W1, b1, W2, b2, W_sh)


def _out_body(p0_ref, p1_ref, xn_ref, W_out_ref, b_out_ref, o_ref):
    acc = p0_ref[...] + p1_ref[...] + xn_ref[...]
    o_ref[...] = jnp.dot(acc, W_out_ref[...], preferred_element_type=jnp.float32) + b_out_ref[...]


def _out_call(part, xn, W_out, b_out):
    full = lambda r, c_: pl.BlockSpec((r, c_), lambda i: (0, 0))
    return pl.pallas_call(
        _out_body,
        grid=(N // _NB,),
        in_specs=[
            pl.BlockSpec((_NB, C), lambda i: (i, 0)),
            pl.BlockSpec((_NB, C), lambda i: (i + N // _NB, 0)),
            pl.BlockSpec((_NB, C), lambda i: (i, 0)),
            full(C, C), full(1, C),
        ],
        out_specs=pl.BlockSpec((_NB, C), lambda i: (i, 0)),
        out_shape=jax.ShapeDtypeStruct((N, C), jnp.float32),
    )(part, part, xn, W_out, b_out)


def kernel(x, edge_index, edge_attr, edge_sh, W_pre, b_pre, Wg1, bg1, Wg2,
           bg2, W_node, b_node, W1, b1, W2, b2, Wl1, bl1, Wl2, bl2, W_sh,
           W_out, b_out):
    dst = edge_index[0]
    src = edge_index[1]
    WlA = Wl1[:C]
    WlB = Wl1[C:]

    pad = EP - E
    spread = jnp.arange(pad, dtype=dst.dtype)
    srcp = jnp.concatenate([src, spread % N]).reshape(ECH, K)
    dstA = jnp.concatenate([dst, spread % N]).reshape(ECH, K)
    dstB = jnp.concatenate([dst, N + (spread % 128)]).reshape(ECH_B, KB)

    pre_x, xn = _node_call(
        x, W_pre, b_pre.reshape(1, C),
        Wg1, bg1.reshape(1, C), Wg2, bg2.reshape(1, C),
        W_node, b_node.reshape(1, C))

    pre_d, prod = _sc_gather(pre_x, dstA, srcp)

    g = _edge_call(pre_d, prod, edge_attr, edge_sh, WlA, WlB,
                   bl1.reshape(1, H), Wl2, bl2.reshape(1, C),
                   W1, b1.reshape(1, H), W2, b2.reshape(1, C), W_sh)

    part = _sc_scatter(g, xn, dstB, srcp)

    return _out_call(part, xn, W_out, b_out.reshape(1, C))


# speedup vs baseline: 1.1418x; 1.0114x over previous
import functools

import jax
import jax.numpy as jnp
from jax import lax
from jax.experimental import pallas as pl
from jax.experimental.pallas import tpu as pltpu
from jax.experimental.pallas import tpu_sc as plsc

N = 10000
E = 160000
C = 128
EA = 16
SH = 25
H = 32

NC = 2
NS = 16
L = 16
NW = NC * NS
K = 128
TRIPS = 40
PAIRS = TRIPS // 2
ECH = NW * TRIPS
EP = ECH * K
KB = 64
TRIPS_B = EP // (KB * NW)
PAIRS_B = TRIPS_B // 2
ECH_B = NW * TRIPS_B
RPT = 624

_mesh = plsc.VectorSubcoreMesh(core_axis_name="c", subcore_axis_name="s")


def _mul_inplace(acc, other, rows):
    def row(r, carry):
        for cc in range(C // L):
            sl = pl.ds(cc * L, L)
            acc[r, sl] = acc[r, sl] * other[r, sl]
        return carry
    lax.fori_loop(0, rows, row, 0)


@functools.partial(
    pl.kernel,
    out_type=(
        jax.ShapeDtypeStruct((EP, C), jnp.float32),
        jax.ShapeDtypeStruct((EP, C), jnp.float32),
    ),
    mesh=_mesh,
    scratch_types=[
        pltpu.VMEM((TRIPS, K), jnp.int32),
        pltpu.VMEM((TRIPS, K), jnp.int32),
        pltpu.VMEM((K, C), jnp.float32),
        pltpu.VMEM((K, C), jnp.float32),
        pltpu.VMEM((K, C), jnp.float32),
        pltpu.VMEM((K, C), jnp.float32),
        pltpu.SemaphoreType.DMA,
        pltpu.SemaphoreType.DMA,
        pltpu.SemaphoreType.DMA,
        pltpu.SemaphoreType.DMA,
        pltpu.SemaphoreType.DMA,
        pltpu.SemaphoreType.DMA,
        pltpu.SemaphoreType.DMA,
        pltpu.SemaphoreType.DMA,
    ],
)
def _sc_gather(pre_hbm, dst_hbm, src_hbm, pred_hbm, prod_hbm,
               dsts, srcs, bufd0, bufs0, bufd1, bufs1,
               gd0, gs0, gd1, gs1, wd0, ws0, wd1, ws1):
    wid = lax.axis_index("s") * NC + lax.axis_index("c")
    w0 = wid * TRIPS
    pltpu.sync_copy(dst_hbm.at[pl.ds(w0, TRIPS)], dsts)
    pltpu.sync_copy(src_hbm.at[pl.ds(w0, TRIPS)], srcs)

    def issue(t, bufd, bufs, gd, gs):
        pltpu.async_copy(pre_hbm.at[dsts.at[t]], bufd, gd)
        pltpu.async_copy(pre_hbm.at[srcs.at[t]], bufs, gs)

    def wait_in(t, bufd, bufs, gd, gs):
        pltpu.make_async_copy(pre_hbm.at[dsts.at[t]], bufd, gd).wait()
        pltpu.make_async_copy(pre_hbm.at[srcs.at[t]], bufs, gs).wait()

    def wait_out(buf, sem):
        pltpu.make_async_copy(pred_hbm.at[pl.ds(0, K)], buf, sem).wait()

    issue(0, bufd0, bufs0, gd0, gs0)
    issue(1, bufd1, bufs1, gd1, gs1)

    def pair(i, carry):
        t0 = 2 * i
        t1 = t0 + 1
        wait_in(t0, bufd0, bufs0, gd0, gs0)
        pltpu.async_copy(bufd0, pred_hbm.at[pl.ds((w0 + t0) * K, K)], wd0)
        _mul_inplace(bufs0, bufd0, K)
        pltpu.async_copy(bufs0, prod_hbm.at[pl.ds((w0 + t0) * K, K)], ws0)
        wait_in(t1, bufd1, bufs1, gd1, gs1)
        pltpu.async_copy(bufd1, pred_hbm.at[pl.ds((w0 + t1) * K, K)], wd1)

        @pl.when(i < PAIRS - 1)
        def _():
            wait_out(bufd0, wd0)
            wait_out(bufs0, ws0)
            issue(t0 + 2, bufd0, bufs0, gd0, gs0)

        _mul_inplace(bufs1, bufd1, K)
        pltpu.async_copy(bufs1, prod_hbm.at[pl.ds((w0 + t1) * K, K)], ws1)

        @pl.when(i < PAIRS - 1)
        def _():
            wait_out(bufd1, wd1)
            wait_out(bufs1, ws1)
            issue(t1 + 2, bufd1, bufs1, gd1, gs1)

        return carry

    lax.fori_loop(0, PAIRS, pair, 0)
    wait_out(bufd0, wd0)
    wait_out(bufs0, ws0)
    wait_out(bufd1, wd1)
    wait_out(bufs1, ws1)


@functools.partial(
    pl.kernel,
    out_type=jax.ShapeDtypeStruct((NC * N, C), jnp.float32),
    mesh=_mesh,
    scratch_types=[
        pltpu.VMEM((TRIPS_B, KB), jnp.int32),
        pltpu.VMEM((TRIPS_B // 2, K), jnp.int32),
        pltpu.VMEM((KB, C), jnp.float32),
        pltpu.VMEM((KB, C), jnp.float32),
        pltpu.VMEM((KB, C), jnp.float32),
        pltpu.VMEM((KB, C), jnp.float32),
        pltpu.VMEM_SHARED((N + 128, C), jnp.float32),
        pltpu.SemaphoreType.DMA,
        pltpu.SemaphoreType.DMA,
        pltpu.SemaphoreType.DMA,
        pltpu.SemaphoreType.DMA,
        pltpu.SemaphoreType.DMA,
        pltpu.SemaphoreType.DMA,
    ],
)
def _sc_scatter(g_hbm, xn_hbm, dst_hbm, src_hbm, out_hbm,
                dsts, srcs, bufx0, bufg0, bufx1, bufg1, agg,
                gx0, gg0, gx1, gg1, ss0, ss1):
    c = lax.axis_index("c")
    s = lax.axis_index("s")
    wid = s * NC + c
    w0 = wid * TRIPS_B
    pltpu.sync_copy(dst_hbm.at[pl.ds(w0, TRIPS_B)], dsts)
    pltpu.sync_copy(src_hbm.at[pl.ds(wid * (TRIPS_B // 2), TRIPS_B // 2)], srcs)

    def zrow(r, carry):
        for cc in range(C // L):
            bufx0[r, pl.ds(cc * L, L)] = jnp.zeros((L,), jnp.float32)
        return carry
    lax.fori_loop(0, KB, zrow, 0)
    for j in range(9):
        pltpu.sync_copy(bufx0, agg.at[pl.ds(s * RPT + j * KB, KB)])
    pltpu.sync_copy(bufx0.at[pl.ds(0, RPT - 9 * KB)],
                    agg.at[pl.ds(s * RPT + 9 * KB, RPT - 9 * KB)])

    @pl.when(s == NS - 1)
    def _():
        pltpu.sync_copy(bufx0, agg.at[pl.ds(NS * RPT, KB)])
        pltpu.sync_copy(bufx0, agg.at[pl.ds(NS * RPT + KB, KB)])
        pltpu.sync_copy(bufx0.at[pl.ds(0, 16)],
                        agg.at[pl.ds(NS * RPT + 2 * KB, 16)])

    plsc.subcore_barrier()

    def issue(t, row, col, bufx, bufg, gx, gg):
        pltpu.async_copy(xn_hbm.at[srcs.at[row, pl.ds(col, KB)]], bufx, gx)
        pltpu.async_copy(g_hbm.at[pl.ds((w0 + t) * KB, KB)], bufg, gg)

    def wait_in(t, row, col, bufx, bufg, gx, gg):
        pltpu.make_async_copy(xn_hbm.at[srcs.at[row, pl.ds(col, KB)]], bufx, gx).wait()
        pltpu.make_async_copy(g_hbm.at[pl.ds((w0 + t) * KB, KB)], bufg, gg).wait()

    def wait_sc(buf, sem):
        pltpu.make_async_copy(g_hbm.at[pl.ds(0, KB)], buf, sem).wait()

    issue(0, 0, 0, bufx0, bufg0, gx0, gg0)
    issue(1, 0, KB, bufx1, bufg1, gx1, gg1)

    def pair(i, carry):
        t0 = 2 * i
        t1 = t0 + 1
        wait_in(t0, i, 0, bufx0, bufg0, gx0, gg0)
        _mul_inplace(bufx0, bufg0, KB)
        pltpu.async_copy(bufx0, agg.at[dsts.at[t0]], ss0, add=True)
        wait_in(t1, i, KB, bufx1, bufg1, gx1, gg1)

        @pl.when(i < PAIRS_B - 1)
        def _():
            wait_sc(bufx0, ss0)
            issue(t0 + 2, i + 1, 0, bufx0, bufg0, gx0, gg0)

        _mul_inplace(bufx1, bufg1, KB)
        pltpu.async_copy(bufx1, agg.at[dsts.at[t1]], ss1, add=True)

        @pl.when(i < PAIRS_B - 1)
        def _():
            wait_sc(bufx1, ss1)
            issue(t1 + 2, i + 1, KB, bufx1, bufg1, gx1, gg1)

        return carry

    lax.fori_loop(0, PAIRS_B, pair, 0)
    wait_sc(bufx0, ss0)
    wait_sc(bufx1, ss1)

    plsc.subcore_barrier()
    pltpu.sync_copy(agg.at[pl.ds(s * RPT, RPT)],
                    out_hbm.at[pl.ds(c * N + s * RPT, RPT)])

    @pl.when(s == NS - 1)
    def _():
        pltpu.sync_copy(agg.at[pl.ds(NS * RPT, N - NS * RPT)],
                        out_hbm.at[pl.ds(c * N + NS * RPT, N - NS * RPT)])


def _node_body(x_ref, W_pre_ref, b_pre_ref, Wg1_ref,
               bg1_ref, Wg2_ref, bg2_ref, W_node_ref, b_node_ref,
               pre_ref, xn_ref):
    x = x_ref[...]
    pre = jnp.dot(x, W_pre_ref[...], preferred_element_type=jnp.float32) + b_pre_ref[...]
    pre_ref[...] = pre
    h1 = jnp.dot(x, Wg1_ref[...], preferred_element_type=jnp.float32) + bg1_ref[...]
    h = jnp.dot(jax.nn.silu(h1), Wg2_ref[...], preferred_element_type=jnp.float32) + bg2_ref[...]
    xn_ref[...] = jnp.dot(x * h, W_node_ref[...], preferred_element_type=jnp.float32) + b_node_ref[...]


_NB = 2000


def _node_call(x, W_pre, b_pre, Wg1, bg1, Wg2, bg2, W_node, b_node):
    full = lambda r, c_: pl.BlockSpec((r, c_), lambda i: (0, 0))
    blk = lambda c_: pl.BlockSpec((_NB, c_), lambda i: (i, 0))
    return pl.pallas_call(
        _node_body,
        grid=(N // _NB,),
        in_specs=[
            blk(C),
            full(C, C), full(1, C),
            full(C, C), full(1, C), full(C, C), full(1, C),
            full(C, C), full(1, C),
        ],
        out_specs=[blk(C), blk(C)],
        out_shape=[
            jax.ShapeDtypeStruct((N, C), jnp.float32),
            jax.ShapeDtypeStruct((N, C), jnp.float32),
        ],
    )(x, W_pre, b_pre, Wg1, bg1, Wg2, bg2, W_node, b_node)


def _edge_body(pred_ref, prod_ref, ea_ref, sh_ref, WlA_ref, WlB_ref, bl1_ref,
               Wl2_ref, bl2_ref, W1_ref, b1_ref, W2_ref, b2_ref, Wsh_ref,
               g_ref):
    u = jax.nn.silu(jnp.dot(ea_ref[...], W1_ref[...], preferred_element_type=jnp.float32) + b1_ref[...])
    w_r = jnp.dot(u, W2_ref[...], preferred_element_type=jnp.float32) + b2_ref[...]
    t = (jnp.dot(pred_ref[...], WlA_ref[...], preferred_element_type=jnp.float32)
         + jnp.dot(prod_ref[...], WlB_ref[...], preferred_element_type=jnp.float32)
         + bl1_ref[...])
    w_s = jnp.dot(jax.nn.silu(t), Wl2_ref[...], preferred_element_type=jnp.float32) + bl2_ref[...]
    sh_p = jnp.dot(sh_ref[...], Wsh_ref[...], preferred_element_type=jnp.float32)
    g_ref[...] = w_r * w_s * sh_p


_EB = 8000


def _edge_call(pre_d, prod, edge_attr, edge_sh, WlA, WlB, bl1, Wl2, bl2,
               W1, b1, W2, b2, W_sh):
    full = lambda r, c_: pl.BlockSpec((r, c_), lambda i: (0, 0))
    blk = lambda c_: pl.BlockSpec((_EB, c_), lambda i: (i, 0))
    return pl.pallas_call(
        _edge_body,
        grid=(E // _EB,),
        in_specs=[
            blk(C), blk(C), blk(EA), blk(SH),
            full(C, H), full(C, H), full(1, H),
            full(H, C), full(1, C),
            full(EA, H), full(1, H), full(H, C), full(1, C),
            full(SH, C),
        ],
        out_specs=blk(C),
        out_shape=jax.ShapeDtypeStruct((EP, C), jnp.float32),
    )(pre_d, prod, edge_attr, edge_sh, WlA, WlB, bl1, Wl2, bl2,
      W1, b1, W2, b2, W_sh)


def _out_body(p0_ref, p1_ref, xn_ref, W_out_ref, b_out_ref, o_ref):
    acc = p0_ref[...] + p1_ref[...] + xn_ref[...]
    o_ref[...] = jnp.dot(acc, W_out_ref[...], preferred_element_type=jnp.float32) + b_out_ref[...]


def _out_call(part, xn, W_out, b_out):
    full = lambda r, c_: pl.BlockSpec((r, c_), lambda i: (0, 0))
    return pl.pallas_call(
        _out_body,
        grid=(N // _NB,),
        in_specs=[
            pl.BlockSpec((_NB, C), lambda i: (i, 0)),
            pl.BlockSpec((_NB, C), lambda i: (i + N // _NB, 0)),
            pl.BlockSpec((_NB, C), lambda i: (i, 0)),
            full(C, C), full(1, C),
        ],
        out_specs=pl.BlockSpec((_NB, C), lambda i: (i, 0)),
        out_shape=jax.ShapeDtypeStruct((N, C), jnp.float32),
    )(part, part, xn, W_out, b_out)


def kernel(x, edge_index, edge_attr, edge_sh, W_pre, b_pre, Wg1, bg1, Wg2,
           bg2, W_node, b_node, W1, b1, W2, b2, Wl1, bl1, Wl2, bl2, W_sh,
           W_out, b_out):
    dst = edge_index[0]
    src = edge_index[1]
    WlA = Wl1[:C]
    WlB = Wl1[C:]

    pad = EP - E
    spread = jnp.arange(pad, dtype=dst.dtype)
    srcp = jnp.concatenate([src, spread % N]).reshape(ECH, K)
    dstA = jnp.concatenate([dst, spread % N]).reshape(ECH, K)
    dstB = jnp.concatenate([dst, N + (spread % 128)]).reshape(ECH_B, KB)

    pre_x, xn = _node_call(
        x, W_pre, b_pre.reshape(1, C),
        Wg1, bg1.reshape(1, C), Wg2, bg2.reshape(1, C),
        W_node, b_node.reshape(1, C))

    pre_d, prod = _sc_gather(pre_x, dstA, srcp)

    g = _edge_call(pre_d, prod, edge_attr, edge_sh, WlA, WlB,
                   bl1.reshape(1, H), Wl2, bl2.reshape(1, C),
                   W1, b1.reshape(1, H), W2, b2.reshape(1, C), W_sh)

    part = _sc_scatter(g, xn, dstB, srcp)

    return _out_call(part, xn, W_out, b_out.reshape(1, C))


# speedup vs baseline: 1.1465x; 1.0041x over previous
import functools

import jax
import jax.numpy as jnp
from jax import lax
from jax.experimental import pallas as pl
from jax.experimental.pallas import tpu as pltpu
from jax.experimental.pallas import tpu_sc as plsc

N = 10000
E = 160000
C = 128
EA = 16
SH = 25
H = 32

NC = 2
NS = 16
L = 16
NW = NC * NS
K = 128
TRIPS = 40
PAIRS = TRIPS // 2
ECH = NW * TRIPS
EP = ECH * K
KB = 64
TRIPS_B = EP // (KB * NW)
PAIRS_B = TRIPS_B // 2
ECH_B = NW * TRIPS_B
RPT = 624

_mesh = plsc.VectorSubcoreMesh(core_axis_name="c", subcore_axis_name="s")


def _mul_inplace(acc, other, rows):
    def row(r, carry):
        for cc in range(C // L):
            sl = pl.ds(cc * L, L)
            acc[r, sl] = acc[r, sl] * other[r, sl]
        return carry
    lax.fori_loop(0, rows, row, 0)


@functools.partial(
    pl.kernel,
    out_type=(
        jax.ShapeDtypeStruct((EP, C), jnp.float32),
        jax.ShapeDtypeStruct((EP, C), jnp.float32),
    ),
    mesh=_mesh,
    scratch_types=[
        pltpu.VMEM((TRIPS, K), jnp.int32),
        pltpu.VMEM((TRIPS, K), jnp.int32),
        pltpu.VMEM((K, C), jnp.float32),
        pltpu.VMEM((K, C), jnp.float32),
        pltpu.VMEM((K, C), jnp.float32),
        pltpu.VMEM((K, C), jnp.float32),
        pltpu.SemaphoreType.DMA,
        pltpu.SemaphoreType.DMA,
        pltpu.SemaphoreType.DMA,
        pltpu.SemaphoreType.DMA,
        pltpu.SemaphoreType.DMA,
        pltpu.SemaphoreType.DMA,
        pltpu.SemaphoreType.DMA,
        pltpu.SemaphoreType.DMA,
    ],
)
def _sc_gather(pre_hbm, dst_hbm, src_hbm, pred_hbm, prod_hbm,
               dsts, srcs, bufd0, bufs0, bufd1, bufs1,
               gd0, gs0, gd1, gs1, wd0, ws0, wd1, ws1):
    wid = lax.axis_index("s") * NC + lax.axis_index("c")
    w0 = wid * TRIPS
    pltpu.sync_copy(dst_hbm.at[pl.ds(w0, TRIPS)], dsts)
    pltpu.sync_copy(src_hbm.at[pl.ds(w0, TRIPS)], srcs)

    def issue(t, bufd, bufs, gd, gs):
        pltpu.async_copy(pre_hbm.at[dsts.at[t]], bufd, gd)
        pltpu.async_copy(pre_hbm.at[srcs.at[t]], bufs, gs)

    def wait_in(t, bufd, bufs, gd, gs):
        pltpu.make_async_copy(pre_hbm.at[dsts.at[t]], bufd, gd).wait()
        pltpu.make_async_copy(pre_hbm.at[srcs.at[t]], bufs, gs).wait()

    def wait_out(buf, sem):
        pltpu.make_async_copy(pred_hbm.at[pl.ds(0, K)], buf, sem).wait()

    issue(0, bufd0, bufs0, gd0, gs0)
    issue(1, bufd1, bufs1, gd1, gs1)

    def pair(i, carry):
        t0 = 2 * i
        t1 = t0 + 1
        wait_in(t0, bufd0, bufs0, gd0, gs0)
        pltpu.async_copy(bufd0, pred_hbm.at[pl.ds((w0 + t0) * K, K)], wd0)
        _mul_inplace(bufs0, bufd0, K)
        pltpu.async_copy(bufs0, prod_hbm.at[pl.ds((w0 + t0) * K, K)], ws0)
        wait_in(t1, bufd1, bufs1, gd1, gs1)
        pltpu.async_copy(bufd1, pred_hbm.at[pl.ds((w0 + t1) * K, K)], wd1)

        @pl.when(i < PAIRS - 1)
        def _():
            wait_out(bufd0, wd0)
            wait_out(bufs0, ws0)
            issue(t0 + 2, bufd0, bufs0, gd0, gs0)

        _mul_inplace(bufs1, bufd1, K)
        pltpu.async_copy(bufs1, prod_hbm.at[pl.ds((w0 + t1) * K, K)], ws1)

        @pl.when(i < PAIRS - 1)
        def _():
            wait_out(bufd1, wd1)
            wait_out(bufs1, ws1)
            issue(t1 + 2, bufd1, bufs1, gd1, gs1)

        return carry

    lax.fori_loop(0, PAIRS, pair, 0)
    wait_out(bufd0, wd0)
    wait_out(bufs0, ws0)
    wait_out(bufd1, wd1)
    wait_out(bufs1, ws1)


@functools.partial(
    pl.kernel,
    out_type=jax.ShapeDtypeStruct((NC * N, C), jnp.float32),
    mesh=_mesh,
    scratch_types=[
        pltpu.VMEM((TRIPS_B, KB), jnp.int32),
        pltpu.VMEM((TRIPS_B // 2, K), jnp.int32),
        pltpu.VMEM((KB, C), jnp.float32),
        pltpu.VMEM((KB, C), jnp.float32),
        pltpu.VMEM((KB, C), jnp.float32),
        pltpu.VMEM((KB, C), jnp.float32),
        pltpu.VMEM_SHARED((N + 128, C), jnp.float32),
        pltpu.SemaphoreType.DMA,
        pltpu.SemaphoreType.DMA,
        pltpu.SemaphoreType.DMA,
        pltpu.SemaphoreType.DMA,
        pltpu.SemaphoreType.DMA,
        pltpu.SemaphoreType.DMA,
    ],
)
def _sc_scatter(g_hbm, xn_hbm, dst_hbm, src_hbm, out_hbm,
                dsts, srcs, bufx0, bufg0, bufx1, bufg1, agg,
                gx0, gg0, gx1, gg1, ss0, ss1):
    c = lax.axis_index("c")
    s = lax.axis_index("s")
    wid = s * NC + c
    w0 = wid * TRIPS_B
    pltpu.sync_copy(dst_hbm.at[pl.ds(w0, TRIPS_B)], dsts)
    pltpu.sync_copy(src_hbm.at[pl.ds(wid * (TRIPS_B // 2), TRIPS_B // 2)], srcs)

    def zrow(r, carry):
        for cc in range(C // L):
            bufx0[r, pl.ds(cc * L, L)] = jnp.zeros((L,), jnp.float32)
        return carry
    lax.fori_loop(0, KB, zrow, 0)
    for j in range(9):
        pltpu.sync_copy(bufx0, agg.at[pl.ds(s * RPT + j * KB, KB)])
    pltpu.sync_copy(bufx0.at[pl.ds(0, RPT - 9 * KB)],
                    agg.at[pl.ds(s * RPT + 9 * KB, RPT - 9 * KB)])

    @pl.when(s == NS - 1)
    def _():
        pltpu.sync_copy(bufx0, agg.at[pl.ds(NS * RPT, KB)])
        pltpu.sync_copy(bufx0, agg.at[pl.ds(NS * RPT + KB, KB)])
        pltpu.sync_copy(bufx0.at[pl.ds(0, 16)],
                        agg.at[pl.ds(NS * RPT + 2 * KB, 16)])

    plsc.subcore_barrier()

    def issue(t, row, col, bufx, bufg, gx, gg):
        pltpu.async_copy(xn_hbm.at[srcs.at[row, pl.ds(col, KB)]], bufx, gx)
        pltpu.async_copy(g_hbm.at[pl.ds((w0 + t) * KB, KB)], bufg, gg)

    def wait_in(t, row, col, bufx, bufg, gx, gg):
        pltpu.make_async_copy(xn_hbm.at[srcs.at[row, pl.ds(col, KB)]], bufx, gx).wait()
        pltpu.make_async_copy(g_hbm.at[pl.ds((w0 + t) * KB, KB)], bufg, gg).wait()

    def wait_sc(buf, sem):
        pltpu.make_async_copy(g_hbm.at[pl.ds(0, KB)], buf, sem).wait()

    issue(0, 0, 0, bufx0, bufg0, gx0, gg0)
    issue(1, 0, KB, bufx1, bufg1, gx1, gg1)

    def pair(i, carry):
        t0 = 2 * i
        t1 = t0 + 1
        wait_in(t0, i, 0, bufx0, bufg0, gx0, gg0)
        _mul_inplace(bufx0, bufg0, KB)
        pltpu.async_copy(bufx0, agg.at[dsts.at[t0]], ss0, add=True)
        wait_in(t1, i, KB, bufx1, bufg1, gx1, gg1)

        @pl.when(i < PAIRS_B - 1)
        def _():
            wait_sc(bufx0, ss0)
            issue(t0 + 2, i + 1, 0, bufx0, bufg0, gx0, gg0)

        _mul_inplace(bufx1, bufg1, KB)
        pltpu.async_copy(bufx1, agg.at[dsts.at[t1]], ss1, add=True)

        @pl.when(i < PAIRS_B - 1)
        def _():
            wait_sc(bufx1, ss1)
            issue(t1 + 2, i + 1, KB, bufx1, bufg1, gx1, gg1)

        return carry

    lax.fori_loop(0, PAIRS_B, pair, 0)
    wait_sc(bufx0, ss0)
    wait_sc(bufx1, ss1)

    plsc.subcore_barrier()
    pltpu.sync_copy(agg.at[pl.ds(s * RPT, RPT)],
                    out_hbm.at[pl.ds(c * N + s * RPT, RPT)])

    @pl.when(s == NS - 1)
    def _():
        pltpu.sync_copy(agg.at[pl.ds(NS * RPT, N - NS * RPT)],
                        out_hbm.at[pl.ds(c * N + NS * RPT, N - NS * RPT)])


def _node_body(x_ref, W_pre_ref, b_pre_ref, Wg1_ref,
               bg1_ref, Wg2_ref, bg2_ref, W_node_ref, b_node_ref,
               pre_ref, xn_ref):
    x = x_ref[...]
    pre = jnp.dot(x, W_pre_ref[...], preferred_element_type=jnp.float32) + b_pre_ref[...]
    pre_ref[...] = pre
    h1 = jnp.dot(x, Wg1_ref[...], preferred_element_type=jnp.float32) + bg1_ref[...]
    h = jnp.dot(jax.nn.silu(h1), Wg2_ref[...], preferred_element_type=jnp.float32) + bg2_ref[...]
    xn_ref[...] = jnp.dot(x * h, W_node_ref[...], preferred_element_type=jnp.float32) + b_node_ref[...]


_NB = 5000


def _node_call(x, W_pre, b_pre, Wg1, bg1, Wg2, bg2, W_node, b_node):
    full = lambda r, c_: pl.BlockSpec((r, c_), lambda i: (0, 0))
    blk = lambda c_: pl.BlockSpec((_NB, c_), lambda i: (i, 0))
    return pl.pallas_call(
        _node_body,
        grid=(N // _NB,),
        in_specs=[
            blk(C),
            full(C, C), full(1, C),
            full(C, C), full(1, C), full(C, C), full(1, C),
            full(C, C), full(1, C),
        ],
        out_specs=[blk(C), blk(C)],
        out_shape=[
            jax.ShapeDtypeStruct((N, C), jnp.float32),
            jax.ShapeDtypeStruct((N, C), jnp.float32),
        ],
    )(x, W_pre, b_pre, Wg1, bg1, Wg2, bg2, W_node, b_node)


def _edge_body(pred_ref, prod_ref, ea_ref, sh_ref, WlA_ref, WlB_ref, bl1_ref,
               Wl2_ref, bl2_ref, W1_ref, b1_ref, W2_ref, b2_ref, Wsh_ref,
               g_ref):
    u = jax.nn.silu(jnp.dot(ea_ref[...], W1_ref[...], preferred_element_type=jnp.float32) + b1_ref[...])
    w_r = jnp.dot(u, W2_ref[...], preferred_element_type=jnp.float32) + b2_ref[...]
    t = (jnp.dot(pred_ref[...], WlA_ref[...], preferred_element_type=jnp.float32)
         + jnp.dot(prod_ref[...], WlB_ref[...], preferred_element_type=jnp.float32)
         + bl1_ref[...])
    w_s = jnp.dot(jax.nn.silu(t), Wl2_ref[...], preferred_element_type=jnp.float32) + bl2_ref[...]
    sh_p = jnp.dot(sh_ref[...], Wsh_ref[...], preferred_element_type=jnp.float32)
    g_ref[...] = w_r * w_s * sh_p


_EB = 10000


def _edge_call(pre_d, prod, edge_attr, edge_sh, WlA, WlB, bl1, Wl2, bl2,
               W1, b1, W2, b2, W_sh):
    full = lambda r, c_: pl.BlockSpec((r, c_), lambda i: (0, 0))
    blk = lambda c_: pl.BlockSpec((_EB, c_), lambda i: (i, 0))
    return pl.pallas_call(
        _edge_body,
        grid=(E // _EB,),
        in_specs=[
            blk(C), blk(C), blk(EA), blk(SH),
            full(C, H), full(C, H), full(1, H),
            full(H, C), full(1, C),
            full(EA, H), full(1, H), full(H, C), full(1, C),
            full(SH, C),
        ],
        out_specs=blk(C),
        out_shape=jax.ShapeDtypeStruct((EP, C), jnp.float32),
    )(pre_d, prod, edge_attr, edge_sh, WlA, WlB, bl1, Wl2, bl2,
      W1, b1, W2, b2, W_sh)


def _out_body(p0_ref, p1_ref, xn_ref, W_out_ref, b_out_ref, o_ref):
    acc = p0_ref[...] + p1_ref[...] + xn_ref[...]
    o_ref[...] = jnp.dot(acc, W_out_ref[...], preferred_element_type=jnp.float32) + b_out_ref[...]


def _out_call(part, xn, W_out, b_out):
    full = lambda r, c_: pl.BlockSpec((r, c_), lambda i: (0, 0))
    return pl.pallas_call(
        _out_body,
        grid=(N // _NB,),
        in_specs=[
            pl.BlockSpec((_NB, C), lambda i: (i, 0)),
            pl.BlockSpec((_NB, C), lambda i: (i + N // _NB, 0)),
            pl.BlockSpec((_NB, C), lambda i: (i, 0)),
            full(C, C), full(1, C),
        ],
        out_specs=pl.BlockSpec((_NB, C), lambda i: (i, 0)),
        out_shape=jax.ShapeDtypeStruct((N, C), jnp.float32),
    )(part, part, xn, W_out, b_out)


def kernel(x, edge_index, edge_attr, edge_sh, W_pre, b_pre, Wg1, bg1, Wg2,
           bg2, W_node, b_node, W1, b1, W2, b2, Wl1, bl1, Wl2, bl2, W_sh,
           W_out, b_out):
    dst = edge_index[0]
    src = edge_index[1]
    WlA = Wl1[:C]
    WlB = Wl1[C:]

    pad = EP - E
    spread = jnp.arange(pad, dtype=dst.dtype)
    srcp = jnp.concatenate([src, spread % N]).reshape(ECH, K)
    dstA = jnp.concatenate([dst, spread % N]).reshape(ECH, K)
    dstB = jnp.concatenate([dst, N + (spread % 128)]).reshape(ECH_B, KB)

    pre_x, xn = _node_call(
        x, W_pre, b_pre.reshape(1, C),
        Wg1, bg1.reshape(1, C), Wg2, bg2.reshape(1, C),
        W_node, b_node.reshape(1, C))

    pre_d, prod = _sc_gather(pre_x, dstA, srcp)

    g = _edge_call(pre_d, prod, edge_attr, edge_sh, WlA, WlB,
                   bl1.reshape(1, H), Wl2, bl2.reshape(1, C),
                   W1, b1.reshape(1, H), W2, b2.reshape(1, C), W_sh)

    part = _sc_scatter(g, xn, dstB, srcp)

    return _out_call(part, xn, W_out, b_out.reshape(1, C))


# speedup vs baseline: 1.1799x; 1.0291x over previous
import functools

import jax
import jax.numpy as jnp
from jax import lax
from jax.experimental import pallas as pl
from jax.experimental.pallas import tpu as pltpu
from jax.experimental.pallas import tpu_sc as plsc

N = 10000
E = 160000
C = 128
EA = 16
SH = 25
H = 32

NC = 2
NS = 16
L = 16
NW = NC * NS
K = 128
TRIPS = 40
PAIRS = TRIPS // 2
ECH = NW * TRIPS
EP = ECH * K
KB = 64
TRIPS_B = EP // (KB * NW)
PAIRS_B = TRIPS_B // 2
ECH_B = NW * TRIPS_B
RPT = 624

_mesh = plsc.VectorSubcoreMesh(core_axis_name="c", subcore_axis_name="s")


def _mul_inplace(acc, other, rows):
    def row(r, carry):
        for cc in range(C // L):
            sl = pl.ds(cc * L, L)
            acc[r, sl] = acc[r, sl] * other[r, sl]
        return carry
    lax.fori_loop(0, rows, row, 0)


@functools.partial(
    pl.kernel,
    out_type=(
        jax.ShapeDtypeStruct((EP, C), jnp.float32),
        jax.ShapeDtypeStruct((EP, C), jnp.float32),
    ),
    mesh=_mesh,
    scratch_types=[
        pltpu.VMEM((TRIPS, K), jnp.int32),
        pltpu.VMEM((TRIPS, K), jnp.int32),
        pltpu.VMEM((K, C), jnp.float32),
        pltpu.VMEM((K, C), jnp.float32),
        pltpu.VMEM((K, C), jnp.float32),
        pltpu.VMEM((K, C), jnp.float32),
        pltpu.SemaphoreType.DMA,
        pltpu.SemaphoreType.DMA,
        pltpu.SemaphoreType.DMA,
        pltpu.SemaphoreType.DMA,
        pltpu.SemaphoreType.DMA,
        pltpu.SemaphoreType.DMA,
        pltpu.SemaphoreType.DMA,
        pltpu.SemaphoreType.DMA,
    ],
)
def _sc_gather(pre_hbm, dst_hbm, src_hbm, pred_hbm, prod_hbm,
               dsts, srcs, bufd0, bufs0, bufd1, bufs1,
               gd0, gs0, gd1, gs1, wd0, ws0, wd1, ws1):
    wid = lax.axis_index("s") * NC + lax.axis_index("c")
    w0 = wid * TRIPS
    pltpu.sync_copy(dst_hbm.at[pl.ds(w0, TRIPS)], dsts)
    pltpu.sync_copy(src_hbm.at[pl.ds(w0, TRIPS)], srcs)

    def issue(t, bufd, bufs, gd, gs):
        pltpu.async_copy(pre_hbm.at[dsts.at[t]], bufd, gd)
        pltpu.async_copy(pre_hbm.at[srcs.at[t]], bufs, gs)

    def wait_in(t, bufd, bufs, gd, gs):
        pltpu.make_async_copy(pre_hbm.at[dsts.at[t]], bufd, gd).wait()
        pltpu.make_async_copy(pre_hbm.at[srcs.at[t]], bufs, gs).wait()

    def wait_out(buf, sem):
        pltpu.make_async_copy(pred_hbm.at[pl.ds(0, K)], buf, sem).wait()

    issue(0, bufd0, bufs0, gd0, gs0)
    issue(1, bufd1, bufs1, gd1, gs1)

    def pair(i, carry):
        t0 = 2 * i
        t1 = t0 + 1
        wait_in(t0, bufd0, bufs0, gd0, gs0)
        pltpu.async_copy(bufd0, pred_hbm.at[pl.ds((w0 + t0) * K, K)], wd0)
        _mul_inplace(bufs0, bufd0, K)
        pltpu.async_copy(bufs0, prod_hbm.at[pl.ds((w0 + t0) * K, K)], ws0)
        wait_in(t1, bufd1, bufs1, gd1, gs1)
        pltpu.async_copy(bufd1, pred_hbm.at[pl.ds((w0 + t1) * K, K)], wd1)

        @pl.when(i < PAIRS - 1)
        def _():
            wait_out(bufd0, wd0)
            wait_out(bufs0, ws0)
            issue(t0 + 2, bufd0, bufs0, gd0, gs0)

        _mul_inplace(bufs1, bufd1, K)
        pltpu.async_copy(bufs1, prod_hbm.at[pl.ds((w0 + t1) * K, K)], ws1)

        @pl.when(i < PAIRS - 1)
        def _():
            wait_out(bufd1, wd1)
            wait_out(bufs1, ws1)
            issue(t1 + 2, bufd1, bufs1, gd1, gs1)

        return carry

    lax.fori_loop(0, PAIRS, pair, 0)
    wait_out(bufd0, wd0)
    wait_out(bufs0, ws0)
    wait_out(bufd1, wd1)
    wait_out(bufs1, ws1)


@functools.partial(
    pl.kernel,
    out_type=jax.ShapeDtypeStruct((NC * N, C), jnp.float32),
    mesh=_mesh,
    scratch_types=[
        pltpu.VMEM((TRIPS_B, KB), jnp.int32),
        pltpu.VMEM((TRIPS_B // 2, K), jnp.int32),
        pltpu.VMEM((KB, C), jnp.float32),
        pltpu.VMEM((KB, C), jnp.float32),
        pltpu.VMEM((KB, C), jnp.float32),
        pltpu.VMEM((KB, C), jnp.float32),
        pltpu.VMEM_SHARED((N + 128, C), jnp.float32),
        pltpu.SemaphoreType.DMA,
        pltpu.SemaphoreType.DMA,
        pltpu.SemaphoreType.DMA,
        pltpu.SemaphoreType.DMA,
        pltpu.SemaphoreType.DMA,
        pltpu.SemaphoreType.DMA,
    ],
)
def _sc_scatter(g_hbm, xn_hbm, dst_hbm, src_hbm, out_hbm,
                dsts, srcs, bufx0, bufg0, bufx1, bufg1, agg,
                gx0, gg0, gx1, gg1, ss0, ss1):
    c = lax.axis_index("c")
    s = lax.axis_index("s")
    wid = s * NC + c
    w0 = wid * TRIPS_B
    pltpu.sync_copy(dst_hbm.at[pl.ds(w0, TRIPS_B)], dsts)
    pltpu.sync_copy(src_hbm.at[pl.ds(wid * (TRIPS_B // 2), TRIPS_B // 2)], srcs)

    def zrow(r, carry):
        for cc in range(C // L):
            bufx0[r, pl.ds(cc * L, L)] = jnp.zeros((L,), jnp.float32)
        return carry
    lax.fori_loop(0, KB, zrow, 0)
    for j in range(9):
        pltpu.sync_copy(bufx0, agg.at[pl.ds(s * RPT + j * KB, KB)])
    pltpu.sync_copy(bufx0.at[pl.ds(0, RPT - 9 * KB)],
                    agg.at[pl.ds(s * RPT + 9 * KB, RPT - 9 * KB)])

    @pl.when(s == NS - 1)
    def _():
        pltpu.sync_copy(bufx0, agg.at[pl.ds(NS * RPT, KB)])
        pltpu.sync_copy(bufx0, agg.at[pl.ds(NS * RPT + KB, KB)])
        pltpu.sync_copy(bufx0.at[pl.ds(0, 16)],
                        agg.at[pl.ds(NS * RPT + 2 * KB, 16)])

    plsc.subcore_barrier()

    def issue(t, row, col, bufx, bufg, gx, gg):
        pltpu.async_copy(xn_hbm.at[srcs.at[row, pl.ds(col, KB)]], bufx, gx)
        pltpu.async_copy(g_hbm.at[pl.ds((w0 + t) * KB, KB)], bufg, gg)

    def wait_in(t, row, col, bufx, bufg, gx, gg):
        pltpu.make_async_copy(xn_hbm.at[srcs.at[row, pl.ds(col, KB)]], bufx, gx).wait()
        pltpu.make_async_copy(g_hbm.at[pl.ds((w0 + t) * KB, KB)], bufg, gg).wait()

    def wait_sc(buf, sem):
        pltpu.make_async_copy(g_hbm.at[pl.ds(0, KB)], buf, sem).wait()

    issue(0, 0, 0, bufx0, bufg0, gx0, gg0)
    issue(1, 0, KB, bufx1, bufg1, gx1, gg1)

    def pair(i, carry):
        t0 = 2 * i
        t1 = t0 + 1
        wait_in(t0, i, 0, bufx0, bufg0, gx0, gg0)
        _mul_inplace(bufx0, bufg0, KB)
        pltpu.async_copy(bufx0, agg.at[dsts.at[t0]], ss0, add=True)
        wait_in(t1, i, KB, bufx1, bufg1, gx1, gg1)

        @pl.when(i < PAIRS_B - 1)
        def _():
            wait_sc(bufx0, ss0)
            issue(t0 + 2, i + 1, 0, bufx0, bufg0, gx0, gg0)

        _mul_inplace(bufx1, bufg1, KB)
        pltpu.async_copy(bufx1, agg.at[dsts.at[t1]], ss1, add=True)

        @pl.when(i < PAIRS_B - 1)
        def _():
            wait_sc(bufx1, ss1)
            issue(t1 + 2, i + 1, KB, bufx1, bufg1, gx1, gg1)

        return carry

    lax.fori_loop(0, PAIRS_B, pair, 0)
    wait_sc(bufx0, ss0)
    wait_sc(bufx1, ss1)

    plsc.subcore_barrier()
    pltpu.sync_copy(agg.at[pl.ds(s * RPT, RPT)],
                    out_hbm.at[pl.ds(c * N + s * RPT, RPT)])

    @pl.when(s == NS - 1)
    def _():
        pltpu.sync_copy(agg.at[pl.ds(NS * RPT, N - NS * RPT)],
                        out_hbm.at[pl.ds(c * N + NS * RPT, N - NS * RPT)])


def _node_body(x_ref, W_pre_ref, b_pre_ref, Wg1_ref,
               bg1_ref, Wg2_ref, bg2_ref, W_node_ref, b_node_ref,
               pre_ref, xn_ref):
    x = x_ref[...]
    pre = jnp.dot(x, W_pre_ref[...], preferred_element_type=jnp.float32) + b_pre_ref[...]
    pre_ref[...] = pre
    h1 = jnp.dot(x, Wg1_ref[...], preferred_element_type=jnp.float32) + bg1_ref[...]
    h = jnp.dot(jax.nn.silu(h1), Wg2_ref[...], preferred_element_type=jnp.float32) + bg2_ref[...]
    xn_ref[...] = jnp.dot(x * h, W_node_ref[...], preferred_element_type=jnp.float32) + b_node_ref[...]


_NB = 5000


def _node_call(x, W_pre, b_pre, Wg1, bg1, Wg2, bg2, W_node, b_node):
    full = lambda r, c_: pl.BlockSpec((r, c_), lambda i: (0, 0))
    blk = lambda c_: pl.BlockSpec((_NB, c_), lambda i: (i, 0))
    return pl.pallas_call(
        _node_body,
        grid=(N // _NB,),
        in_specs=[
            blk(C),
            full(C, C), full(1, C),
            full(C, C), full(1, C), full(C, C), full(1, C),
            full(C, C), full(1, C),
        ],
        out_specs=[blk(C), blk(C)],
        out_shape=[
            jax.ShapeDtypeStruct((N, C), jnp.float32),
            jax.ShapeDtypeStruct((N, C), jnp.float32),
        ],
    )(x, W_pre, b_pre, Wg1, bg1, Wg2, bg2, W_node, b_node)


def _edge_body(pred_ref, prod_ref, ea_ref, sh_ref, WlA_ref, WlB_ref, bl1_ref,
               Wl2_ref, bl2_ref, W1_ref, b1_ref, W2_ref, b2_ref, Wsh_ref,
               g_ref):
    u = jax.nn.silu(jnp.dot(ea_ref[...], W1_ref[...], preferred_element_type=jnp.float32) + b1_ref[...])
    w_r = jnp.dot(u, W2_ref[...], preferred_element_type=jnp.float32) + b2_ref[...]
    t = (jnp.dot(pred_ref[...], WlA_ref[...], preferred_element_type=jnp.float32)
         + jnp.dot(prod_ref[...], WlB_ref[...], preferred_element_type=jnp.float32)
         + bl1_ref[...])
    w_s = jnp.dot(jax.nn.silu(t), Wl2_ref[...], preferred_element_type=jnp.float32) + bl2_ref[...]
    sh_p = jnp.dot(sh_ref[...], Wsh_ref[...], preferred_element_type=jnp.float32)
    g_ref[...] = w_r * w_s * sh_p


_EB = 8000


def _edge_call(pre_d, prod, edge_attr, edge_sh, WlA, WlB, bl1, Wl2, bl2,
               W1, b1, W2, b2, W_sh):
    full = lambda r, c_: pl.BlockSpec((r, c_), lambda i: (0, 0))
    blk = lambda c_: pl.BlockSpec((_EB, c_), lambda i: (i, 0))
    return pl.pallas_call(
        _edge_body,
        grid=(E // _EB,),
        in_specs=[
            blk(C), blk(C), blk(EA), blk(SH),
            full(C, H), full(C, H), full(1, H),
            full(H, C), full(1, C),
            full(EA, H), full(1, H), full(H, C), full(1, C),
            full(SH, C),
        ],
        out_specs=blk(C),
        out_shape=jax.ShapeDtypeStruct((EP, C), jnp.float32),
    )(pre_d, prod, edge_attr, edge_sh, WlA, WlB, bl1, Wl2, bl2,
      W1, b1, W2, b2, W_sh)


def _out_body(p0_ref, p1_ref, xn_ref, W_out_ref, b_out_ref, o_ref):
    acc = p0_ref[...] + p1_ref[...] + xn_ref[...]
    o_ref[...] = jnp.dot(acc, W_out_ref[...], preferred_element_type=jnp.float32) + b_out_ref[...]


def _out_call(part, xn, W_out, b_out):
    full = lambda r, c_: pl.BlockSpec((r, c_), lambda i: (0, 0))
    return pl.pallas_call(
        _out_body,
        grid=(N // _NB,),
        in_specs=[
            pl.BlockSpec((_NB, C), lambda i: (i, 0)),
            pl.BlockSpec((_NB, C), lambda i: (i + N // _NB, 0)),
            pl.BlockSpec((_NB, C), lambda i: (i, 0)),
            full(C, C), full(1, C),
        ],
        out_specs=pl.BlockSpec((_NB, C), lambda i: (i, 0)),
        out_shape=jax.ShapeDtypeStruct((N, C), jnp.float32),
    )(part, part, xn, W_out, b_out)


def kernel(x, edge_index, edge_attr, edge_sh, W_pre, b_pre, Wg1, bg1, Wg2,
           bg2, W_node, b_node, W1, b1, W2, b2, Wl1, bl1, Wl2, bl2, W_sh,
           W_out, b_out):
    dst = edge_index[0]
    src = edge_index[1]
    WlA = Wl1[:C]
    WlB = Wl1[C:]

    pad = EP - E
    spread = jnp.arange(pad, dtype=dst.dtype)
    srcp = jnp.concatenate([src, spread % N]).reshape(ECH, K)
    dstA = jnp.concatenate([dst, spread % N]).reshape(ECH, K)
    dstB = jnp.concatenate([dst, N + (spread % 128)]).reshape(ECH_B, KB)

    pre_x, xn = _node_call(
        x, W_pre, b_pre.reshape(1, C),
        Wg1, bg1.reshape(1, C), Wg2, bg2.reshape(1, C),
        W_node, b_node.reshape(1, C))

    pre_d, prod = _sc_gather(pre_x, dstA, srcp)

    g = _edge_call(pre_d, prod, edge_attr, edge_sh, WlA, WlB,
                   bl1.reshape(1, H), Wl2, bl2.reshape(1, C),
                   W1, b1.reshape(1, H), W2, b2.reshape(1, C), W_sh)

    part = _sc_scatter(g, xn, dstB, srcp)

    return _out_call(part, xn, W_out, b_out.reshape(1, C))
